# baseline (speedup 1.0000x reference)
import functools

import jax
import jax.numpy as jnp
from jax import lax
from jax.experimental import pallas as pl
from jax.experimental.pallas import tpu as pltpu
from jax.experimental.pallas import tpu_sc as plsc

N = 10000
NP = 10240
E = 320000
D = 128
HOPS = 3
NC = 2
NS = 16
NW = NC * NS
CH = 128
NCH = 84
EW = NCH * CH
EP = NW * EW
RT = NP // NS
BN = 512
BM = 400

_mesh = plsc.VectorSubcoreMesh(core_axis_name="c", subcore_axis_name="s")


@functools.partial(
    pl.kernel,
    out_type=jax.ShapeDtypeStruct((NC, NP), jnp.float32),
    mesh=_mesh,
    scratch_types=[
        pltpu.VMEM((4, CH), jnp.int32),
        pltpu.VMEM((CH,), jnp.float32),
        pltpu.VMEM((RT,), jnp.float32),
        pltpu.VMEM_SHARED((NP,), jnp.float32),
        pltpu.SemaphoreType.DMA,
        pltpu.SemaphoreType.DMA,
        pltpu.SemaphoreType.DMA,
        pltpu.SemaphoreType.DMA,
        pltpu.SemaphoreType.DMA,
        pltpu.SemaphoreType.DMA,
        pltpu.SemaphoreType.DMA,
        pltpu.SemaphoreType.DMA,
    ],
)
def _deg_kernel(col_hbm, out_hbm, idxw, ones_v, stage_v, acc,
                is0, is1, is2, is3, ss0, ss1, ss2, ss3):
    c = lax.axis_index("c")
    s = lax.axis_index("s")
    wid = c * NS + s
    iss = (is0, is1, is2, is3)
    sss = (ss0, ss1, ss2, ss3)
    for i in range(CH // 16):
        ones_v[pl.ds(i * 16, 16)] = jnp.ones((16,), jnp.float32)
    for i in range(RT // 16):
        stage_v[pl.ds(i * 16, 16)] = jnp.zeros((16,), jnp.float32)
    pltpu.sync_copy(stage_v, acc.at[pl.ds(s * RT, RT)])
    plsc.subcore_barrier()

    for p in range(2):
        pltpu.make_async_copy(col_hbm.at[wid, p], idxw.at[p], iss[p]).start()

    def body(t, carry):
        for p in range(4):
            j = t * 4 + p
            pltpu.make_async_copy(col_hbm.at[wid, j], idxw.at[p],
                                  iss[p]).wait()
            pltpu.async_copy(ones_v, acc.at[idxw.at[p]], sss[p], add=True)
            pn = (p + 2) % 4

            @pl.when((j >= 2) & (j + 2 < NCH))
            def _():
                pltpu.make_async_copy(ones_v, acc.at[idxw.at[pn]],
                                      sss[pn]).wait()

            @pl.when(j + 2 < NCH)
            def _():
                pltpu.make_async_copy(col_hbm.at[wid, j + 2], idxw.at[pn],
                                      iss[pn]).start()
        return carry

    lax.fori_loop(0, NCH // 4, body, 0)
    for p in range(4):
        pltpu.make_async_copy(ones_v, acc.at[idxw.at[p]], sss[p]).wait()
    plsc.subcore_barrier()
    pltpu.sync_copy(acc.at[pl.ds(s * RT, RT)], stage_v)
    pltpu.sync_copy(stage_v, out_hbm.at[c, pl.ds(s * RT, RT)])


@functools.partial(
    pl.kernel,
    out_type=jax.ShapeDtypeStruct((NC, NP, D), jnp.float32),
    mesh=_mesh,
    scratch_types=[
        pltpu.VMEM((4, CH), jnp.int32),
        pltpu.VMEM((4, CH), jnp.int32),
        pltpu.VMEM((CH, D), jnp.float32),
        pltpu.VMEM((CH, D), jnp.float32),
        pltpu.VMEM_SHARED((NP, D), jnp.float32),
        pltpu.SemaphoreType.DMA,
        pltpu.SemaphoreType.DMA,
        pltpu.SemaphoreType.DMA,
        pltpu.SemaphoreType.DMA,
        pltpu.SemaphoreType.DMA,
        pltpu.SemaphoreType.DMA,
        pltpu.SemaphoreType.DMA,
        pltpu.SemaphoreType.DMA,
        pltpu.SemaphoreType.DMA,
        pltpu.SemaphoreType.DMA,
        pltpu.SemaphoreType.DMA,
        pltpu.SemaphoreType.DMA,
        pltpu.SemaphoreType.DMA,
        pltpu.SemaphoreType.DMA,
        pltpu.SemaphoreType.DMA,
        pltpu.SemaphoreType.DMA,
    ],
)
def _hop_kernel(y_hbm, col_hbm, row_hbm, out_hbm,
                idxgw, idxsw, r0, r1, acc,
                ig0, ig1, ig2, ig3, is0, is1, is2, is3,
                ss0, ss1, ss2, ss3, gs0, gs1, ws0, ws1):
    c = lax.axis_index("c")
    s = lax.axis_index("s")
    wid = c * NS + s
    rows = (r0, r1)
    igs = (ig0, ig1, ig2, ig3)
    iss = (is0, is1, is2, is3)
    sss = (ss0, ss1, ss2, ss3)
    gss = (gs0, gs1)

    for i in range(CH):
        for k in range(D // 16):
            r0[i, pl.ds(k * 16, 16)] = jnp.zeros((16,), jnp.float32)
    for i in range(RT // CH):
        pltpu.sync_copy(r0, acc.at[pl.ds(s * RT + i * CH, CH)])
    plsc.subcore_barrier()

    for p in range(4):
        pltpu.make_async_copy(col_hbm.at[wid, p], idxgw.at[p], igs[p]).start()
    for p in range(3):
        pltpu.make_async_copy(row_hbm.at[wid, p], idxsw.at[p], iss[p]).start()
    pltpu.make_async_copy(col_hbm.at[wid, 0], idxgw.at[0], igs[0]).wait()
    pltpu.make_async_copy(y_hbm.at[idxgw.at[0]], rows[0], gss[0]).start()

    def body(t, carry):
        for p in range(4):
            j = t * 4 + p
            r = p % 2
            pltpu.make_async_copy(y_hbm.at[idxgw.at[p]], rows[r],
                                  gss[r]).wait()

            @pl.when(j + 4 < NCH)
            def _():
                pltpu.make_async_copy(col_hbm.at[wid, j + 4], idxgw.at[p],
                                      igs[p]).start()
            pltpu.make_async_copy(row_hbm.at[wid, j], idxsw.at[p],
                                  iss[p]).wait()
            pltpu.async_copy(rows[r], acc.at[idxsw.at[p]], sss[p], add=True)
            pn = (p + 3) % 4

            @pl.when((j >= 1) & (j + 1 < NCH))
            def _():
                pltpu.make_async_copy(rows[1 - r], acc.at[idxsw.at[pn]],
                                      sss[pn]).wait()

            @pl.when(j + 3 < NCH)
            def _():
                pltpu.make_async_copy(row_hbm.at[wid, j + 3], idxsw.at[pn],
                                      iss[pn]).start()

            @pl.when(j + 1 < NCH)
            def _():
                pp = (p + 1) % 4
                pltpu.make_async_copy(col_hbm.at[wid, j + 1], idxgw.at[pp],
                                      igs[pp]).wait()
                pltpu.make_async_copy(y_hbm.at[idxgw.at[pp]], rows[1 - r],
                                      gss[1 - r]).start()
        return carry

    lax.fori_loop(0, NCH // 4, body, 0)
    pltpu.make_async_copy(rows[0], acc.at[idxsw.at[2]], sss[2]).wait()
    pltpu.make_async_copy(rows[1], acc.at[idxsw.at[3]], sss[3]).wait()
    plsc.subcore_barrier()

    sts = (r0, r1)
    wss = (ws0, ws1)
    descs = [None, None]
    for i in range(RT // CH):
        b = i % 2
        if descs[b] is not None:
            descs[b].wait()
        pltpu.sync_copy(acc.at[pl.ds(s * RT + i * CH, CH)], sts[b])
        descs[b] = pltpu.async_copy(
            sts[b], out_hbm.at[c, pl.ds(s * RT + i * CH, CH)], wss[b])
    for d_ in descs:
        d_.wait()


def _norm_body(degp_ref, x_ref, y_ref, dis_ref, dgi_ref):
    deg = degp_ref[0] + degp_ref[1]
    dis = lax.rsqrt(deg)
    y_ref[...] = x_ref[...] * dis
    dis_ref[...] = dis
    dgi_ref[...] = 1.0 / deg


_norm = pl.pallas_call(
    _norm_body,
    grid=(NP // BN,),
    in_specs=[
        pl.BlockSpec((NC, BN, 1), lambda i: (0, i, 0)),
        pl.BlockSpec((BN, D), lambda i: (i, 0)),
    ],
    out_specs=[
        pl.BlockSpec((BN, D), lambda i: (i, 0)),
        pl.BlockSpec((BN, 1), lambda i: (i, 0)),
        pl.BlockSpec((BN, 1), lambda i: (i, 0)),
    ],
    out_shape=[
        jax.ShapeDtypeStruct((NP, D), jnp.float32),
        jax.ShapeDtypeStruct((NP, 1), jnp.float32),
        jax.ShapeDtypeStruct((NP, 1), jnp.float32),
    ],
)


def _scale_body(zp_ref, dgi_ref, y_ref):
    y_ref[...] = (zp_ref[0] + zp_ref[1]) * dgi_ref[...]


_scale = pl.pallas_call(
    _scale_body,
    grid=(NP // BN,),
    in_specs=[
        pl.BlockSpec((NC, BN, D), lambda i: (0, i, 0)),
        pl.BlockSpec((BN, 1), lambda i: (i, 0)),
    ],
    out_specs=pl.BlockSpec((BN, D), lambda i: (i, 0)),
    out_shape=jax.ShapeDtypeStruct((NP, D), jnp.float32),
)


def _final_body(zp_ref, dis_ref, w_ref, b_ref, o_ref):
    z = (zp_ref[0] + zp_ref[1]) * dis_ref[...]
    o_ref[...] = lax.dot_general(
        z, w_ref[...], (((1,), (1,)), ((), ())),
        preferred_element_type=jnp.float32) + b_ref[...]


_final = pl.pallas_call(
    _final_body,
    grid=(N // BM,),
    in_specs=[
        pl.BlockSpec((NC, BM, D), lambda i: (0, i, 0)),
        pl.BlockSpec((BM, 1), lambda i: (i, 0)),
        pl.BlockSpec((D, D), lambda i: (0, 0)),
        pl.BlockSpec((1, D), lambda i: (0, 0)),
    ],
    out_specs=pl.BlockSpec((BM, D), lambda i: (i, 0)),
    out_shape=jax.ShapeDtypeStruct((N, D), jnp.float32),
)


@jax.jit
def kernel(x, edge_index, W, b):
    row = edge_index[0]
    col = edge_index[1]
    loop = jnp.arange(N, dtype=jnp.int32)
    padv = N + (jnp.arange(EP - E - N, dtype=jnp.int32) % (NP - N))
    colp = jnp.concatenate([col, loop, padv]).reshape(NW, NCH, CH)
    rowp = jnp.concatenate([row, loop, padv]).reshape(NW, NCH, CH)
    xp = jnp.pad(x, ((0, NP - N), (0, 0)))

    degp = _deg_kernel(colp)
    y, dis, dgi = _norm(degp.reshape(NC, NP, 1), xp)
    for _ in range(HOPS - 1):
        zp = _hop_kernel(y, colp, rowp)
        y = _scale(zp, dgi)
    zp = _hop_kernel(y, colp, rowp)
    return _final(zp, dis, W, b.reshape(1, D))


# speedup vs baseline: 1.2559x; 1.2559x over previous
import functools

import jax
import jax.numpy as jnp
from jax import lax
from jax.experimental import pallas as pl
from jax.experimental.pallas import tpu as pltpu
from jax.experimental.pallas import tpu_sc as plsc

N = 10000
NP = 10240
E = 320000
D = 128
HOPS = 3
NC = 2
NS = 16
NW = NC * NS
CH = 128
NCH = 82
EW = NCH * CH
EP = NW * EW
RT = NP // NS
BN = 1024
BM = 1000

_mesh = plsc.VectorSubcoreMesh(core_axis_name="c", subcore_axis_name="s")


@functools.partial(
    pl.kernel,
    out_type=jax.ShapeDtypeStruct((NC, NP), jnp.float32),
    mesh=_mesh,
    scratch_types=[
        pltpu.VMEM((2, CH), jnp.int32),
        pltpu.VMEM((CH,), jnp.float32),
        pltpu.VMEM((RT,), jnp.float32),
        pltpu.VMEM_SHARED((NP,), jnp.float32),
        pltpu.SemaphoreType.DMA,
        pltpu.SemaphoreType.DMA,
        pltpu.SemaphoreType.DMA,
        pltpu.SemaphoreType.DMA,
    ],
)
def _deg_kernel(er_hbm, out_hbm, idxw, ones_v, stage_v, acc,
                is0, is1, ss0, ss1):
    c = lax.axis_index("c")
    s = lax.axis_index("s")
    wid = c * NS + s
    iss = (is0, is1)
    sss = (ss0, ss1)
    for i in range(CH // 16):
        ones_v[pl.ds(i * 16, 16)] = jnp.ones((16,), jnp.float32)
    for i in range(RT // 16):
        stage_v[pl.ds(i * 16, 16)] = jnp.zeros((16,), jnp.float32)
    pltpu.sync_copy(stage_v, acc.at[pl.ds(s * RT, RT)])
    plsc.subcore_barrier()

    for b in range(2):
        pltpu.make_async_copy(er_hbm.at[1, wid, b], idxw.at[b], iss[b]).start()

    def body(t, carry):
        for b in range(2):
            j = t * 2 + b
            pltpu.make_async_copy(er_hbm.at[1, wid, j], idxw.at[b],
                                  iss[b]).wait()
            pltpu.async_copy(ones_v, acc.at[idxw.at[b]], sss[b], add=True)
        for b in range(2):
            j = t * 2 + b

            @pl.when(j + 2 < NCH)
            def _():
                pltpu.make_async_copy(ones_v, acc.at[idxw.at[b]],
                                      sss[b]).wait()
                pltpu.make_async_copy(er_hbm.at[1, wid, j + 2], idxw.at[b],
                                      iss[b]).start()
        return carry

    lax.fori_loop(0, NCH // 2, body, 0)
    for b in range(2):
        pltpu.make_async_copy(ones_v, acc.at[idxw.at[b]], sss[b]).wait()
    plsc.subcore_barrier()
    pltpu.sync_copy(acc.at[pl.ds(s * RT, RT)], stage_v)
    pltpu.sync_copy(stage_v, out_hbm.at[c, pl.ds(s * RT, RT)])


@functools.partial(
    pl.kernel,
    out_type=jax.ShapeDtypeStruct((NC, NP, D), jnp.float32),
    mesh=_mesh,
    scratch_types=[
        pltpu.VMEM((2, CH), jnp.int32),
        pltpu.VMEM((2, CH), jnp.int32),
        pltpu.VMEM((CH, D), jnp.float32),
        pltpu.VMEM((CH, D), jnp.float32),
        pltpu.VMEM_SHARED((NP, D), jnp.float32),
        pltpu.SemaphoreType.DMA,
        pltpu.SemaphoreType.DMA,
        pltpu.SemaphoreType.DMA,
        pltpu.SemaphoreType.DMA,
        pltpu.SemaphoreType.DMA,
        pltpu.SemaphoreType.DMA,
        pltpu.SemaphoreType.DMA,
        pltpu.SemaphoreType.DMA,
    ],
)
def _hop_kernel(y_hbm, er_hbm, out_hbm,
                idxgw, idxsw, r0, r1, acc,
                ig0, ig1, is0, is1, gs0, gs1, ws0, ws1):
    c = lax.axis_index("c")
    s = lax.axis_index("s")
    wid = c * NS + s
    rows = (r0, r1)
    igs = (ig0, ig1)
    iss = (is0, is1)
    gss = (gs0, gs1)

    for i in range(CH):
        for k in range(D // 16):
            r0[i, pl.ds(k * 16, 16)] = jnp.zeros((16,), jnp.float32)
    for i in range(RT // CH):
        pltpu.sync_copy(r0, acc.at[pl.ds(s * RT + i * CH, CH)])
    plsc.subcore_barrier()

    for b in range(2):
        pltpu.make_async_copy(er_hbm.at[1, wid, b], idxgw.at[b], igs[b]).start()
        pltpu.make_async_copy(er_hbm.at[0, wid, b], idxsw.at[b], iss[b]).start()
    for b in range(2):
        pltpu.make_async_copy(er_hbm.at[1, wid, b], idxgw.at[b], igs[b]).wait()
        pltpu.make_async_copy(y_hbm.at[idxgw.at[b]], rows[b], gss[b]).start()

    def body(t, carry):
        for b in range(2):
            j = t * 2 + b
            pltpu.make_async_copy(y_hbm.at[idxgw.at[b]], rows[b],
                                  gss[b]).wait()

            @pl.when(j + 2 < NCH)
            def _():
                pltpu.make_async_copy(er_hbm.at[1, wid, j + 2], idxgw.at[b],
                                      igs[b]).start()
            pltpu.make_async_copy(er_hbm.at[0, wid, j], idxsw.at[b],
                                  iss[b]).wait()
            pltpu.sync_copy(rows[b], acc.at[idxsw.at[b]], add=True)

            @pl.when(j + 2 < NCH)
            def _():
                pltpu.make_async_copy(er_hbm.at[0, wid, j + 2], idxsw.at[b],
                                      iss[b]).start()
                pltpu.make_async_copy(er_hbm.at[1, wid, j + 2], idxgw.at[b],
                                      igs[b]).wait()
                pltpu.make_async_copy(y_hbm.at[idxgw.at[b]], rows[b],
                                      gss[b]).start()
        return carry

    lax.fori_loop(0, NCH // 2, body, 0)
    plsc.subcore_barrier()

    sts = (r0, r1)
    wss = (ws0, ws1)
    descs = [None, None]
    for i in range(RT // CH):
        b = i % 2
        if descs[b] is not None:
            descs[b].wait()
        pltpu.sync_copy(acc.at[pl.ds(s * RT + i * CH, CH)], sts[b])
        descs[b] = pltpu.async_copy(
            sts[b], out_hbm.at[c, pl.ds(s * RT + i * CH, CH)], wss[b])
    for d_ in descs:
        d_.wait()


def _norm_body(degp_ref, x_ref, y_ref, deg_ref):
    deg = degp_ref[0] + degp_ref[1]
    y_ref[...] = x_ref[...] * lax.rsqrt(deg)
    deg_ref[...] = deg


_norm = pl.pallas_call(
    _norm_body,
    grid=(NP // BN,),
    in_specs=[
        pl.BlockSpec((NC, BN, 1), lambda i: (0, i, 0)),
        pl.BlockSpec((BN, D), lambda i: (i, 0)),
    ],
    out_specs=[
        pl.BlockSpec((BN, D), lambda i: (i, 0)),
        pl.BlockSpec((BN, 1), lambda i: (i, 0)),
    ],
    out_shape=[
        jax.ShapeDtypeStruct((NP, D), jnp.float32),
        jax.ShapeDtypeStruct((NP, 1), jnp.float32),
    ],
)


def _scale_body(zp_ref, deg_ref, y_ref):
    y_ref[...] = (zp_ref[0] + zp_ref[1]) / deg_ref[...]


_scale = pl.pallas_call(
    _scale_body,
    grid=(NP // BN,),
    in_specs=[
        pl.BlockSpec((NC, BN, D), lambda i: (0, i, 0)),
        pl.BlockSpec((BN, 1), lambda i: (i, 0)),
    ],
    out_specs=pl.BlockSpec((BN, D), lambda i: (i, 0)),
    out_shape=jax.ShapeDtypeStruct((NP, D), jnp.float32),
)


def _final_body(zp_ref, deg_ref, w_ref, b_ref, o_ref):
    z = (zp_ref[0] + zp_ref[1]) * lax.rsqrt(deg_ref[...])
    o_ref[...] = lax.dot_general(
        z, w_ref[...], (((1,), (1,)), ((), ())),
        preferred_element_type=jnp.float32) + b_ref[...]


_final = pl.pallas_call(
    _final_body,
    grid=(N // BM,),
    in_specs=[
        pl.BlockSpec((NC, BM, D), lambda i: (0, i, 0)),
        pl.BlockSpec((BM, 1), lambda i: (i, 0)),
        pl.BlockSpec((D, D), lambda i: (0, 0)),
        pl.BlockSpec((1, D), lambda i: (0, 0)),
    ],
    out_specs=pl.BlockSpec((BM, D), lambda i: (i, 0)),
    out_shape=jax.ShapeDtypeStruct((N, D), jnp.float32),
)


@jax.jit
def kernel(x, edge_index, W, b):
    loop = jnp.arange(N, dtype=jnp.int32)
    padv = N + (jnp.arange(EP - E - N, dtype=jnp.int32) % (NP - N))
    tail = jnp.tile(jnp.concatenate([loop, padv])[None, :], (2, 1))
    er = jnp.concatenate([edge_index, tail], axis=1).reshape(2, NW, NCH, CH)
    xp = jnp.pad(x, ((0, NP - N), (0, 0)))

    degp = _deg_kernel(er)
    y, deg1 = _norm(degp.reshape(NC, NP, 1), xp)
    for _ in range(HOPS - 1):
        zp = _hop_kernel(y, er)
        y = _scale(zp, deg1)
    zp = _hop_kernel(y, er)
    return _final(zp, deg1, W, b.reshape(1, D))


# speedup vs baseline: 1.2649x; 1.0072x over previous
import functools

import jax
import jax.numpy as jnp
from jax import lax
from jax.experimental import pallas as pl
from jax.experimental.pallas import tpu as pltpu
from jax.experimental.pallas import tpu_sc as plsc

N = 10000
NP = 10240
E = 320000
D = 128
HOPS = 3
NC = 2
NS = 16
NW = NC * NS
CH = 128
NCH = 82
EW = NCH * CH
EP = NW * EW
RT = NP // NS
BN = 1024
BM = 1000

_mesh = plsc.VectorSubcoreMesh(core_axis_name="c", subcore_axis_name="s")


@functools.partial(
    pl.kernel,
    out_type=jax.ShapeDtypeStruct((NC, NP), jnp.float32),
    mesh=_mesh,
    scratch_types=[
        pltpu.VMEM((4, CH), jnp.int32),
        pltpu.VMEM((CH,), jnp.float32),
        pltpu.VMEM((RT,), jnp.float32),
        pltpu.VMEM_SHARED((NP,), jnp.float32),
        pltpu.SemaphoreType.DMA,
        pltpu.SemaphoreType.DMA,
        pltpu.SemaphoreType.DMA,
        pltpu.SemaphoreType.DMA,
        pltpu.SemaphoreType.DMA,
        pltpu.SemaphoreType.DMA,
        pltpu.SemaphoreType.DMA,
        pltpu.SemaphoreType.DMA,
    ],
)
def _deg_kernel(er_hbm, out_hbm, idxw, ones_v, stage_v, acc,
                is0, is1, is2, is3, ss0, ss1, ss2, ss3):
    c = lax.axis_index("c")
    s = lax.axis_index("s")
    wid = c * NS + s
    iss = (is0, is1, is2, is3)
    sss = (ss0, ss1, ss2, ss3)
    for i in range(CH // 16):
        ones_v[pl.ds(i * 16, 16)] = jnp.ones((16,), jnp.float32)
    for i in range(RT // 16):
        stage_v[pl.ds(i * 16, 16)] = jnp.zeros((16,), jnp.float32)
    pltpu.sync_copy(stage_v, acc.at[pl.ds(s * RT, RT)])
    plsc.subcore_barrier()

    for p in range(2):
        pltpu.make_async_copy(er_hbm.at[1, wid, p], idxw.at[p], iss[p]).start()

    NB = (NCH // 4) * 4

    def body(t, carry):
        for p in range(4):
            j = t * 4 + p
            pltpu.make_async_copy(er_hbm.at[1, wid, j], idxw.at[p],
                                  iss[p]).wait()
            pltpu.async_copy(ones_v, acc.at[idxw.at[p]], sss[p], add=True)
            pn = (p + 2) % 4

            @pl.when((j >= 2) & (j + 2 < NCH))
            def _():
                pltpu.make_async_copy(ones_v, acc.at[idxw.at[pn]],
                                      sss[pn]).wait()

            @pl.when(j + 2 < NCH)
            def _():
                pltpu.make_async_copy(er_hbm.at[1, wid, j + 2], idxw.at[pn],
                                      iss[pn]).start()
        return carry

    lax.fori_loop(0, NB // 4, body, 0)
    for j in range(NB, NCH):
        p = j % 4
        pltpu.make_async_copy(er_hbm.at[1, wid, j], idxw.at[p], iss[p]).wait()
        pltpu.async_copy(ones_v, acc.at[idxw.at[p]], sss[p], add=True)
    for j in range(NCH - 4, NCH):
        p = j % 4
        pltpu.make_async_copy(ones_v, acc.at[idxw.at[p]], sss[p]).wait()
    plsc.subcore_barrier()
    pltpu.sync_copy(acc.at[pl.ds(s * RT, RT)], stage_v)
    pltpu.sync_copy(stage_v, out_hbm.at[c, pl.ds(s * RT, RT)])


@functools.partial(
    pl.kernel,
    out_type=jax.ShapeDtypeStruct((NC, NP, D), jnp.float32),
    mesh=_mesh,
    scratch_types=[
        pltpu.VMEM((2, CH), jnp.int32),
        pltpu.VMEM((2, CH), jnp.int32),
        pltpu.VMEM((CH, D), jnp.float32),
        pltpu.VMEM((CH, D), jnp.float32),
        pltpu.VMEM_SHARED((NP, D), jnp.float32),
        pltpu.SemaphoreType.DMA,
        pltpu.SemaphoreType.DMA,
        pltpu.SemaphoreType.DMA,
        pltpu.SemaphoreType.DMA,
        pltpu.SemaphoreType.DMA,
        pltpu.SemaphoreType.DMA,
        pltpu.SemaphoreType.DMA,
        pltpu.SemaphoreType.DMA,
    ],
)
def _hop_kernel(y_hbm, er_hbm, out_hbm,
                idxgw, idxsw, r0, r1, acc,
                ig0, ig1, is0, is1, gs0, gs1, ws0, ws1):
    c = lax.axis_index("c")
    s = lax.axis_index("s")
    wid = c * NS + s
    rows = (r0, r1)
    igs = (ig0, ig1)
    iss = (is0, is1)
    gss = (gs0, gs1)

    for i in range(CH):
        for k in range(D // 16):
            r0[i, pl.ds(k * 16, 16)] = jnp.zeros((16,), jnp.float32)
    for i in range(RT // CH):
        pltpu.sync_copy(r0, acc.at[pl.ds(s * RT + i * CH, CH)])
    plsc.subcore_barrier()

    for b in range(2):
        pltpu.make_async_copy(er_hbm.at[1, wid, b], idxgw.at[b], igs[b]).start()
        pltpu.make_async_copy(er_hbm.at[0, wid, b], idxsw.at[b], iss[b]).start()
    for b in range(2):
        pltpu.make_async_copy(er_hbm.at[1, wid, b], idxgw.at[b], igs[b]).wait()
        pltpu.make_async_copy(y_hbm.at[idxgw.at[b]], rows[b], gss[b]).start()

    def body(t, carry):
        for b in range(2):
            j = t * 2 + b
            pltpu.make_async_copy(y_hbm.at[idxgw.at[b]], rows[b],
                                  gss[b]).wait()

            @pl.when(j + 2 < NCH)
            def _():
                pltpu.make_async_copy(er_hbm.at[1, wid, j + 2], idxgw.at[b],
                                      igs[b]).start()
            pltpu.make_async_copy(er_hbm.at[0, wid, j], idxsw.at[b],
                                  iss[b]).wait()
            pltpu.sync_copy(rows[b], acc.at[idxsw.at[b]], add=True)

            @pl.when(j + 2 < NCH)
            def _():
                pltpu.make_async_copy(er_hbm.at[0, wid, j + 2], idxsw.at[b],
                                      iss[b]).start()
                pltpu.make_async_copy(er_hbm.at[1, wid, j + 2], idxgw.at[b],
                                      igs[b]).wait()
                pltpu.make_async_copy(y_hbm.at[idxgw.at[b]], rows[b],
                                      gss[b]).start()
        return carry

    lax.fori_loop(0, NCH // 2, body, 0)
    plsc.subcore_barrier()

    sts = (r0, r1)
    wss = (ws0, ws1)
    descs = [None, None]
    for i in range(RT // CH):
        b = i % 2
        if descs[b] is not None:
            descs[b].wait()
        pltpu.sync_copy(acc.at[pl.ds(s * RT + i * CH, CH)], sts[b])
        descs[b] = pltpu.async_copy(
            sts[b], out_hbm.at[c, pl.ds(s * RT + i * CH, CH)], wss[b])
    for d_ in descs:
        d_.wait()


def _norm_body(degp_ref, x_ref, y_ref, deg_ref):
    deg = degp_ref[0] + degp_ref[1]
    y_ref[...] = x_ref[...] * lax.rsqrt(deg)
    deg_ref[...] = deg


_norm = pl.pallas_call(
    _norm_body,
    grid=(NP // BN,),
    in_specs=[
        pl.BlockSpec((NC, BN, 1), lambda i: (0, i, 0)),
        pl.BlockSpec((BN, D), lambda i: (i, 0)),
    ],
    out_specs=[
        pl.BlockSpec((BN, D), lambda i: (i, 0)),
        pl.BlockSpec((BN, 1), lambda i: (i, 0)),
    ],
    out_shape=[
        jax.ShapeDtypeStruct((NP, D), jnp.float32),
        jax.ShapeDtypeStruct((NP, 1), jnp.float32),
    ],
)


def _scale_body(zp_ref, deg_ref, y_ref):
    y_ref[...] = (zp_ref[0] + zp_ref[1]) / deg_ref[...]


_scale = pl.pallas_call(
    _scale_body,
    grid=(NP // BN,),
    in_specs=[
        pl.BlockSpec((NC, BN, D), lambda i: (0, i, 0)),
        pl.BlockSpec((BN, 1), lambda i: (i, 0)),
    ],
    out_specs=pl.BlockSpec((BN, D), lambda i: (i, 0)),
    out_shape=jax.ShapeDtypeStruct((NP, D), jnp.float32),
)


def _final_body(zp_ref, deg_ref, w_ref, b_ref, o_ref):
    z = (zp_ref[0] + zp_ref[1]) * lax.rsqrt(deg_ref[...])
    o_ref[...] = lax.dot_general(
        z, w_ref[...], (((1,), (1,)), ((), ())),
        preferred_element_type=jnp.float32) + b_ref[...]


_final = pl.pallas_call(
    _final_body,
    grid=(N // BM,),
    in_specs=[
        pl.BlockSpec((NC, BM, D), lambda i: (0, i, 0)),
        pl.BlockSpec((BM, 1), lambda i: (i, 0)),
        pl.BlockSpec((D, D), lambda i: (0, 0)),
        pl.BlockSpec((1, D), lambda i: (0, 0)),
    ],
    out_specs=pl.BlockSpec((BM, D), lambda i: (i, 0)),
    out_shape=jax.ShapeDtypeStruct((N, D), jnp.float32),
)


@jax.jit
def kernel(x, edge_index, W, b):
    loop = jnp.arange(N, dtype=jnp.int32)
    padv = N + (jnp.arange(EP - E - N, dtype=jnp.int32) % (NP - N))
    tail = jnp.tile(jnp.concatenate([loop, padv])[None, :], (2, 1))
    er = jnp.concatenate([edge_index, tail], axis=1).reshape(2, NW, NCH, CH)
    xp = jnp.pad(x, ((0, NP - N), (0, 0)))

    degp = _deg_kernel(er)
    y, deg1 = _norm(degp.reshape(NC, NP, 1), xp)
    for _ in range(HOPS - 1):
        zp = _hop_kernel(y, er)
        y = _scale(zp, deg1)
    zp = _hop_kernel(y, er)
    return _final(zp, deg1, W, b.reshape(1, D))


# speedup vs baseline: 1.2772x; 1.0097x over previous
import functools

import jax
import jax.numpy as jnp
from jax import lax
from jax.experimental import pallas as pl
from jax.experimental.pallas import tpu as pltpu
from jax.experimental.pallas import tpu_sc as plsc

N = 10000
NP = 10240
E = 320000
D = 128
HOPS = 3
NC = 2
NS = 16
NW = NC * NS
CH = 128
NCH = 82
EW = NCH * CH
EP = NW * EW
RT = NP // NS
EC = E // CH
TC_ = (EP - E) // CH
BN = 1024
BM = 1000

import numpy as _np
_TAIL = _np.concatenate([
    _np.arange(N, dtype=_np.int32),
    N + (_np.arange(EP - E - N, dtype=_np.int32) % (NP - N)),
]).reshape(TC_, CH)

_mesh = plsc.VectorSubcoreMesh(core_axis_name="c", subcore_axis_name="s")


@functools.partial(
    pl.kernel,
    out_type=jax.ShapeDtypeStruct((NC, NP), jnp.float32),
    mesh=_mesh,
    scratch_types=[
        pltpu.VMEM((4, CH), jnp.int32),
        pltpu.VMEM((CH,), jnp.float32),
        pltpu.VMEM((RT,), jnp.float32),
        pltpu.VMEM_SHARED((NP,), jnp.float32),
        pltpu.SemaphoreType.DMA,
        pltpu.SemaphoreType.DMA,
        pltpu.SemaphoreType.DMA,
        pltpu.SemaphoreType.DMA,
        pltpu.SemaphoreType.DMA,
        pltpu.SemaphoreType.DMA,
        pltpu.SemaphoreType.DMA,
        pltpu.SemaphoreType.DMA,
    ],
)
def _deg_kernel(ei_hbm, tail_hbm, out_hbm, idxw, ones_v, stage_v, acc,
                is0, is1, is2, is3, ss0, ss1, ss2, ss3):
    c = lax.axis_index("c")
    s = lax.axis_index("s")
    wid = c * NS + s
    iss = (is0, is1, is2, is3)
    sss = (ss0, ss1, ss2, ss3)

    def ld_col(j, dst, sem):
        q = wid * NCH + j

        @pl.when(q < EC)
        def _():
            pltpu.make_async_copy(ei_hbm.at[1, q], dst, sem).start()

        @pl.when(q >= EC)
        def _():
            pltpu.make_async_copy(tail_hbm.at[q - EC], dst, sem).start()

    def wt_col(j, dst, sem):
        q = jnp.minimum(wid * NCH + j, EC - 1)
        pltpu.make_async_copy(ei_hbm.at[1, q], dst, sem).wait()
    for i in range(CH // 16):
        ones_v[pl.ds(i * 16, 16)] = jnp.ones((16,), jnp.float32)
    for i in range(RT // 16):
        stage_v[pl.ds(i * 16, 16)] = jnp.zeros((16,), jnp.float32)
    pltpu.sync_copy(stage_v, acc.at[pl.ds(s * RT, RT)])
    plsc.subcore_barrier()

    for p in range(2):
        ld_col(p, idxw.at[p], iss[p])

    NB = (NCH // 4) * 4

    def body(t, carry):
        for p in range(4):
            j = t * 4 + p
            wt_col(j, idxw.at[p], iss[p])
            pltpu.async_copy(ones_v, acc.at[idxw.at[p]], sss[p], add=True)
            pn = (p + 2) % 4

            @pl.when((j >= 2) & (j + 2 < NCH))
            def _():
                pltpu.make_async_copy(ones_v, acc.at[idxw.at[pn]],
                                      sss[pn]).wait()

            @pl.when(j + 2 < NCH)
            def _():
                ld_col(j + 2, idxw.at[pn], iss[pn])
        return carry

    lax.fori_loop(0, NB // 4, body, 0)
    for j in range(NB, NCH):
        p = j % 4
        wt_col(j, idxw.at[p], iss[p])
        pltpu.async_copy(ones_v, acc.at[idxw.at[p]], sss[p], add=True)
    for j in range(NCH - 4, NCH):
        p = j % 4
        pltpu.make_async_copy(ones_v, acc.at[idxw.at[p]], sss[p]).wait()
    plsc.subcore_barrier()
    pltpu.sync_copy(acc.at[pl.ds(s * RT, RT)], stage_v)
    pltpu.sync_copy(stage_v, out_hbm.at[c, pl.ds(s * RT, RT)])


@functools.partial(
    pl.kernel,
    out_type=jax.ShapeDtypeStruct((NC, NP, D), jnp.float32),
    mesh=_mesh,
    scratch_types=[
        pltpu.VMEM((2, CH), jnp.int32),
        pltpu.VMEM((2, CH), jnp.int32),
        pltpu.VMEM((CH, D), jnp.float32),
        pltpu.VMEM((CH, D), jnp.float32),
        pltpu.VMEM_SHARED((NP, D), jnp.float32),
        pltpu.SemaphoreType.DMA,
        pltpu.SemaphoreType.DMA,
        pltpu.SemaphoreType.DMA,
        pltpu.SemaphoreType.DMA,
        pltpu.SemaphoreType.DMA,
        pltpu.SemaphoreType.DMA,
        pltpu.SemaphoreType.DMA,
        pltpu.SemaphoreType.DMA,
    ],
)
def _hop_kernel(y_hbm, ei_hbm, tail_hbm, out_hbm,
                idxgw, idxsw, r0, r1, acc,
                ig0, ig1, is0, is1, gs0, gs1, ws0, ws1):
    c = lax.axis_index("c")
    s = lax.axis_index("s")
    wid = c * NS + s
    rows = (r0, r1)
    igs = (ig0, ig1)
    iss = (is0, is1)
    gss = (gs0, gs1)

    def ld_idx(which, j, dst, sem):
        q = wid * NCH + j

        @pl.when(q < EC)
        def _():
            pltpu.make_async_copy(ei_hbm.at[which, q], dst, sem).start()

        @pl.when(q >= EC)
        def _():
            pltpu.make_async_copy(tail_hbm.at[q - EC], dst, sem).start()

    def wt_idx(which, j, dst, sem):
        q = jnp.minimum(wid * NCH + j, EC - 1)
        pltpu.make_async_copy(ei_hbm.at[which, q], dst, sem).wait()

    for i in range(CH):
        for k in range(D // 16):
            r0[i, pl.ds(k * 16, 16)] = jnp.zeros((16,), jnp.float32)
    for i in range(RT // CH):
        pltpu.sync_copy(r0, acc.at[pl.ds(s * RT + i * CH, CH)])
    plsc.subcore_barrier()

    for b in range(2):
        ld_idx(1, b, idxgw.at[b], igs[b])
        ld_idx(0, b, idxsw.at[b], iss[b])
    for b in range(2):
        wt_idx(1, b, idxgw.at[b], igs[b])
        pltpu.make_async_copy(y_hbm.at[idxgw.at[b]], rows[b], gss[b]).start()

    def body(t, carry):
        for b in range(2):
            j = t * 2 + b
            pltpu.make_async_copy(y_hbm.at[idxgw.at[b]], rows[b],
                                  gss[b]).wait()

            @pl.when(j + 2 < NCH)
            def _():
                ld_idx(1, j + 2, idxgw.at[b], igs[b])
            wt_idx(0, j, idxsw.at[b], iss[b])
            pltpu.sync_copy(rows[b], acc.at[idxsw.at[b]], add=True)

            @pl.when(j + 2 < NCH)
            def _():
                ld_idx(0, j + 2, idxsw.at[b], iss[b])
                wt_idx(1, j + 2, idxgw.at[b], igs[b])
                pltpu.make_async_copy(y_hbm.at[idxgw.at[b]], rows[b],
                                      gss[b]).start()
        return carry

    lax.fori_loop(0, NCH // 2, body, 0)
    plsc.subcore_barrier()

    sts = (r0, r1)
    wss = (ws0, ws1)
    descs = [None, None]
    for i in range(RT // CH):
        b = i % 2
        if descs[b] is not None:
            descs[b].wait()
        pltpu.sync_copy(acc.at[pl.ds(s * RT + i * CH, CH)], sts[b])
        descs[b] = pltpu.async_copy(
            sts[b], out_hbm.at[c, pl.ds(s * RT + i * CH, CH)], wss[b])
    for d_ in descs:
        d_.wait()


def _norm_body(degp_ref, x_ref, y_ref, deg_ref):
    deg = degp_ref[0] + degp_ref[1]
    y_ref[...] = x_ref[...] * lax.rsqrt(deg)
    deg_ref[...] = deg


_norm = pl.pallas_call(
    _norm_body,
    grid=(NP // BN,),
    in_specs=[
        pl.BlockSpec((NC, BN, 1), lambda i: (0, i, 0)),
        pl.BlockSpec((BN, D), lambda i: (i, 0)),
    ],
    out_specs=[
        pl.BlockSpec((BN, D), lambda i: (i, 0)),
        pl.BlockSpec((BN, 1), lambda i: (i, 0)),
    ],
    out_shape=[
        jax.ShapeDtypeStruct((NP, D), jnp.float32),
        jax.ShapeDtypeStruct((NP, 1), jnp.float32),
    ],
)


def _scale_body(zp_ref, deg_ref, y_ref):
    y_ref[...] = (zp_ref[0] + zp_ref[1]) / deg_ref[...]


_scale = pl.pallas_call(
    _scale_body,
    grid=(NP // BN,),
    in_specs=[
        pl.BlockSpec((NC, BN, D), lambda i: (0, i, 0)),
        pl.BlockSpec((BN, 1), lambda i: (i, 0)),
    ],
    out_specs=pl.BlockSpec((BN, D), lambda i: (i, 0)),
    out_shape=jax.ShapeDtypeStruct((NP, D), jnp.float32),
)


def _final_body(zp_ref, deg_ref, w_ref, b_ref, o_ref):
    z = (zp_ref[0] + zp_ref[1]) * lax.rsqrt(deg_ref[...])
    o_ref[...] = lax.dot_general(
        z, w_ref[...], (((1,), (1,)), ((), ())),
        preferred_element_type=jnp.float32) + b_ref[...]


_final = pl.pallas_call(
    _final_body,
    grid=(N // BM,),
    in_specs=[
        pl.BlockSpec((NC, BM, D), lambda i: (0, i, 0)),
        pl.BlockSpec((BM, 1), lambda i: (i, 0)),
        pl.BlockSpec((D, D), lambda i: (0, 0)),
        pl.BlockSpec((1, D), lambda i: (0, 0)),
    ],
    out_specs=pl.BlockSpec((BM, D), lambda i: (i, 0)),
    out_shape=jax.ShapeDtypeStruct((N, D), jnp.float32),
)


@jax.jit
def kernel(x, edge_index, W, b):
    ei = edge_index.reshape(2, EC, CH)
    tail = jnp.asarray(_TAIL)

    degp = _deg_kernel(ei, tail)
    y, deg1 = _norm(degp.reshape(NC, NP, 1), x)
    for _ in range(HOPS - 1):
        zp = _hop_kernel(y, ei, tail)
        y = _scale(zp, deg1)
    zp = _hop_kernel(y, ei, tail)
    return _final(zp, deg1, W, b.reshape(1, D))


# speedup vs baseline: 1.2845x; 1.0057x over previous
import functools

import jax
import jax.numpy as jnp
from jax import lax
from jax.experimental import pallas as pl
from jax.experimental.pallas import tpu as pltpu
from jax.experimental.pallas import tpu_sc as plsc

N = 10000
NP = 10240
E = 320000
D = 128
HOPS = 3
NC = 2
NS = 16
NW = NC * NS
CH = 128
NCH = 82
EW = NCH * CH
EP = NW * EW
RT = NP // NS
EC = E // CH
TC_ = (EP - E) // CH
BN = 1024
BM = 1000

import numpy as _np
_TAIL = _np.concatenate([
    _np.arange(N, dtype=_np.int32),
    N + (_np.arange(EP - E - N, dtype=_np.int32) % (NP - N)),
]).reshape(TC_, CH)

_mesh = plsc.VectorSubcoreMesh(core_axis_name="c", subcore_axis_name="s")


@functools.partial(
    pl.kernel,
    out_type=jax.ShapeDtypeStruct((NC, NP), jnp.float32),
    mesh=_mesh,
    scratch_types=[
        pltpu.VMEM((4, CH), jnp.int32),
        pltpu.VMEM((CH,), jnp.float32),
        pltpu.VMEM((RT,), jnp.float32),
        pltpu.VMEM_SHARED((NP,), jnp.float32),
        pltpu.SemaphoreType.DMA,
        pltpu.SemaphoreType.DMA,
        pltpu.SemaphoreType.DMA,
        pltpu.SemaphoreType.DMA,
        pltpu.SemaphoreType.DMA,
        pltpu.SemaphoreType.DMA,
        pltpu.SemaphoreType.DMA,
        pltpu.SemaphoreType.DMA,
    ],
)
def _deg_kernel(ei_hbm, tail_hbm, out_hbm, idxw, ones_v, stage_v, acc,
                is0, is1, is2, is3, ss0, ss1, ss2, ss3):
    c = lax.axis_index("c")
    s = lax.axis_index("s")
    wid = c * NS + s
    iss = (is0, is1, is2, is3)
    sss = (ss0, ss1, ss2, ss3)

    def ld_col(j, dst, sem):
        q = wid * NCH + j

        @pl.when(q < EC)
        def _():
            pltpu.make_async_copy(ei_hbm.at[1, q], dst, sem).start()

        @pl.when(q >= EC)
        def _():
            pltpu.make_async_copy(tail_hbm.at[q - EC], dst, sem).start()

    def wt_col(j, dst, sem):
        q = jnp.minimum(wid * NCH + j, EC - 1)
        pltpu.make_async_copy(ei_hbm.at[1, q], dst, sem).wait()
    for p in range(2):
        ld_col(p, idxw.at[p], iss[p])
    for i in range(CH // 16):
        ones_v[pl.ds(i * 16, 16)] = jnp.ones((16,), jnp.float32)
    for i in range(RT // 16):
        stage_v[pl.ds(i * 16, 16)] = jnp.zeros((16,), jnp.float32)
    pltpu.sync_copy(stage_v, acc.at[pl.ds(s * RT, RT)])
    plsc.subcore_barrier()

    NB = (NCH // 4) * 4

    def body(t, carry):
        for p in range(4):
            j = t * 4 + p
            wt_col(j, idxw.at[p], iss[p])
            pltpu.async_copy(ones_v, acc.at[idxw.at[p]], sss[p], add=True)
            pn = (p + 2) % 4

            @pl.when((j >= 2) & (j + 2 < NCH))
            def _():
                pltpu.make_async_copy(ones_v, acc.at[idxw.at[pn]],
                                      sss[pn]).wait()

            @pl.when(j + 2 < NCH)
            def _():
                ld_col(j + 2, idxw.at[pn], iss[pn])
        return carry

    lax.fori_loop(0, NB // 4, body, 0)
    for j in range(NB, NCH):
        p = j % 4
        wt_col(j, idxw.at[p], iss[p])
        pltpu.async_copy(ones_v, acc.at[idxw.at[p]], sss[p], add=True)
    for j in range(NCH - 4, NCH):
        p = j % 4
        pltpu.make_async_copy(ones_v, acc.at[idxw.at[p]], sss[p]).wait()
    plsc.subcore_barrier()
    pltpu.sync_copy(acc.at[pl.ds(s * RT, RT)], stage_v)
    pltpu.sync_copy(stage_v, out_hbm.at[c, pl.ds(s * RT, RT)])


@functools.partial(
    pl.kernel,
    out_type=jax.ShapeDtypeStruct((NC, NP, D), jnp.float32),
    mesh=_mesh,
    scratch_types=[
        pltpu.VMEM((2, CH), jnp.int32),
        pltpu.VMEM((2, CH), jnp.int32),
        pltpu.VMEM((CH, D), jnp.float32),
        pltpu.VMEM((CH, D), jnp.float32),
        pltpu.VMEM_SHARED((NP, D), jnp.float32),
        pltpu.SemaphoreType.DMA,
        pltpu.SemaphoreType.DMA,
        pltpu.SemaphoreType.DMA,
        pltpu.SemaphoreType.DMA,
        pltpu.SemaphoreType.DMA,
        pltpu.SemaphoreType.DMA,
        pltpu.SemaphoreType.DMA,
        pltpu.SemaphoreType.DMA,
    ],
)
def _hop_kernel(y_hbm, ei_hbm, tail_hbm, out_hbm,
                idxgw, idxsw, r0, r1, acc,
                ig0, ig1, is0, is1, gs0, gs1, ws0, ws1):
    c = lax.axis_index("c")
    s = lax.axis_index("s")
    wid = c * NS + s
    rows = (r0, r1)
    igs = (ig0, ig1)
    iss = (is0, is1)
    gss = (gs0, gs1)

    def ld_idx(which, j, dst, sem):
        q = wid * NCH + j

        @pl.when(q < EC)
        def _():
            pltpu.make_async_copy(ei_hbm.at[which, q], dst, sem).start()

        @pl.when(q >= EC)
        def _():
            pltpu.make_async_copy(tail_hbm.at[q - EC], dst, sem).start()

    def wt_idx(which, j, dst, sem):
        q = jnp.minimum(wid * NCH + j, EC - 1)
        pltpu.make_async_copy(ei_hbm.at[which, q], dst, sem).wait()

    for b in range(2):
        ld_idx(1, b, idxgw.at[b], igs[b])
        ld_idx(0, b, idxsw.at[b], iss[b])
    for i in range(CH):
        for k in range(D // 16):
            r0[i, pl.ds(k * 16, 16)] = jnp.zeros((16,), jnp.float32)
    for i in range(RT // CH):
        pltpu.make_async_copy(r0, acc.at[pl.ds(s * RT + i * CH, CH)],
                              ws0).start()
    for i in range(RT // CH):
        pltpu.make_async_copy(r0, acc.at[pl.ds(s * RT + i * CH, CH)],
                              ws0).wait()
    for b in range(2):
        wt_idx(1, b, idxgw.at[b], igs[b])
        pltpu.make_async_copy(y_hbm.at[idxgw.at[b]], rows[b], gss[b]).start()
    plsc.subcore_barrier()

    def body(t, carry):
        for b in range(2):
            j = t * 2 + b
            pltpu.make_async_copy(y_hbm.at[idxgw.at[b]], rows[b],
                                  gss[b]).wait()

            @pl.when(j + 2 < NCH)
            def _():
                ld_idx(1, j + 2, idxgw.at[b], igs[b])
            wt_idx(0, j, idxsw.at[b], iss[b])
            pltpu.sync_copy(rows[b], acc.at[idxsw.at[b]], add=True)

            @pl.when(j + 2 < NCH)
            def _():
                ld_idx(0, j + 2, idxsw.at[b], iss[b])
                wt_idx(1, j + 2, idxgw.at[b], igs[b])
                pltpu.make_async_copy(y_hbm.at[idxgw.at[b]], rows[b],
                                      gss[b]).start()
        return carry

    lax.fori_loop(0, NCH // 2, body, 0)
    plsc.subcore_barrier()

    sts = (r0, r1)
    wss = (ws0, ws1)
    descs = [None, None]
    for i in range(RT // CH):
        b = i % 2
        if descs[b] is not None:
            descs[b].wait()
        pltpu.sync_copy(acc.at[pl.ds(s * RT + i * CH, CH)], sts[b])
        descs[b] = pltpu.async_copy(
            sts[b], out_hbm.at[c, pl.ds(s * RT + i * CH, CH)], wss[b])
    for d_ in descs:
        d_.wait()


def _norm_body(degp_ref, x_ref, y_ref, deg_ref):
    deg = degp_ref[0] + degp_ref[1]
    y_ref[...] = x_ref[...] * lax.rsqrt(deg)
    deg_ref[...] = deg


_norm = pl.pallas_call(
    _norm_body,
    grid=(NP // BN,),
    in_specs=[
        pl.BlockSpec((NC, BN, 1), lambda i: (0, i, 0)),
        pl.BlockSpec((BN, D), lambda i: (i, 0)),
    ],
    out_specs=[
        pl.BlockSpec((BN, D), lambda i: (i, 0)),
        pl.BlockSpec((BN, 1), lambda i: (i, 0)),
    ],
    out_shape=[
        jax.ShapeDtypeStruct((NP, D), jnp.float32),
        jax.ShapeDtypeStruct((NP, 1), jnp.float32),
    ],
)


def _scale_body(zp_ref, deg_ref, y_ref):
    y_ref[...] = (zp_ref[0] + zp_ref[1]) / deg_ref[...]


_scale = pl.pallas_call(
    _scale_body,
    grid=(NP // BN,),
    in_specs=[
        pl.BlockSpec((NC, BN, D), lambda i: (0, i, 0)),
        pl.BlockSpec((BN, 1), lambda i: (i, 0)),
    ],
    out_specs=pl.BlockSpec((BN, D), lambda i: (i, 0)),
    out_shape=jax.ShapeDtypeStruct((NP, D), jnp.float32),
)


def _final_body(zp_ref, deg_ref, w_ref, b_ref, o_ref):
    z = (zp_ref[0] + zp_ref[1]) * lax.rsqrt(deg_ref[...])
    o_ref[...] = lax.dot_general(
        z, w_ref[...], (((1,), (1,)), ((), ())),
        preferred_element_type=jnp.float32) + b_ref[...]


_final = pl.pallas_call(
    _final_body,
    grid=(N // BM,),
    in_specs=[
        pl.BlockSpec((NC, BM, D), lambda i: (0, i, 0)),
        pl.BlockSpec((BM, 1), lambda i: (i, 0)),
        pl.BlockSpec((D, D), lambda i: (0, 0)),
        pl.BlockSpec((1, D), lambda i: (0, 0)),
    ],
    out_specs=pl.BlockSpec((BM, D), lambda i: (i, 0)),
    out_shape=jax.ShapeDtypeStruct((N, D), jnp.float32),
)


@jax.jit
def kernel(x, edge_index, W, b):
    ei = edge_index.reshape(2, EC, CH)
    tail = jnp.asarray(_TAIL)

    degp = _deg_kernel(ei, tail)
    y, deg1 = _norm(degp.reshape(NC, NP, 1), x)
    for _ in range(HOPS - 1):
        zp = _hop_kernel(y, ei, tail)
        y = _scale(zp, deg1)
    zp = _hop_kernel(y, ei, tail)
    return _final(zp, deg1, W, b.reshape(1, D))


# speedup vs baseline: 1.2992x; 1.0115x over previous
import functools

import jax
import jax.numpy as jnp
from jax import lax
from jax.experimental import pallas as pl
from jax.experimental.pallas import tpu as pltpu
from jax.experimental.pallas import tpu_sc as plsc

N = 10000
NP = 10240
E = 320000
D = 128
HOPS = 3
NC = 2
NS = 16
NW = NC * NS
CH = 128
NCH = 81
EW = NCH * CH
EP = NW * EW
RT = NP // NS
EC = E // CH
TC_ = (EP - E) // CH
BN = 1024
BM = 1000

import numpy as _np
_TAIL = _np.concatenate([
    _np.arange(N, dtype=_np.int32),
    N + (_np.arange(EP - E - N, dtype=_np.int32) % (NP - N)),
]).reshape(TC_, CH)

_mesh = plsc.VectorSubcoreMesh(core_axis_name="c", subcore_axis_name="s")


@functools.partial(
    pl.kernel,
    out_type=jax.ShapeDtypeStruct((NC, NP), jnp.float32),
    mesh=_mesh,
    scratch_types=[
        pltpu.VMEM((4, CH), jnp.int32),
        pltpu.VMEM((CH,), jnp.float32),
        pltpu.VMEM((RT,), jnp.float32),
        pltpu.VMEM_SHARED((NP,), jnp.float32),
        pltpu.SemaphoreType.DMA,
        pltpu.SemaphoreType.DMA,
        pltpu.SemaphoreType.DMA,
        pltpu.SemaphoreType.DMA,
        pltpu.SemaphoreType.DMA,
        pltpu.SemaphoreType.DMA,
        pltpu.SemaphoreType.DMA,
        pltpu.SemaphoreType.DMA,
    ],
)
def _deg_kernel(ei_hbm, tail_hbm, out_hbm, idxw, ones_v, stage_v, acc,
                is0, is1, is2, is3, ss0, ss1, ss2, ss3):
    c = lax.axis_index("c")
    s = lax.axis_index("s")
    wid = c * NS + s
    iss = (is0, is1, is2, is3)
    sss = (ss0, ss1, ss2, ss3)

    def ld_col(j, dst, sem):
        q = wid * NCH + j

        @pl.when(q < EC)
        def _():
            pltpu.make_async_copy(ei_hbm.at[1, q], dst, sem).start()

        @pl.when(q >= EC)
        def _():
            pltpu.make_async_copy(tail_hbm.at[q - EC], dst, sem).start()

    def wt_col(j, dst, sem):
        q = jnp.minimum(wid * NCH + j, EC - 1)
        pltpu.make_async_copy(ei_hbm.at[1, q], dst, sem).wait()
    for p in range(2):
        ld_col(p, idxw.at[p], iss[p])
    for i in range(CH // 16):
        ones_v[pl.ds(i * 16, 16)] = jnp.ones((16,), jnp.float32)
    for i in range(RT // 16):
        stage_v[pl.ds(i * 16, 16)] = jnp.zeros((16,), jnp.float32)
    pltpu.sync_copy(stage_v, acc.at[pl.ds(s * RT, RT)])
    plsc.subcore_barrier()

    NB = (NCH // 4) * 4

    def body(t, carry):
        for p in range(4):
            j = t * 4 + p
            wt_col(j, idxw.at[p], iss[p])
            pltpu.async_copy(ones_v, acc.at[idxw.at[p]], sss[p], add=True)
            pn = (p + 2) % 4

            @pl.when((j >= 2) & (j + 2 < NCH))
            def _():
                pltpu.make_async_copy(ones_v, acc.at[idxw.at[pn]],
                                      sss[pn]).wait()

            @pl.when(j + 2 < NCH)
            def _():
                ld_col(j + 2, idxw.at[pn], iss[pn])
        return carry

    lax.fori_loop(0, NB // 4, body, 0)
    for j in range(NB, NCH):
        p = j % 4
        wt_col(j, idxw.at[p], iss[p])
        pltpu.async_copy(ones_v, acc.at[idxw.at[p]], sss[p], add=True)
    for j in range(NCH - 4, NCH):
        p = j % 4
        pltpu.make_async_copy(ones_v, acc.at[idxw.at[p]], sss[p]).wait()
    plsc.subcore_barrier()
    pltpu.sync_copy(acc.at[pl.ds(s * RT, RT)], stage_v)
    pltpu.sync_copy(stage_v, out_hbm.at[c, pl.ds(s * RT, RT)])


@functools.partial(
    pl.kernel,
    out_type=jax.ShapeDtypeStruct((NC, NP, D), jnp.float32),
    mesh=_mesh,
    scratch_types=[
        pltpu.VMEM((2, CH), jnp.int32),
        pltpu.VMEM((2, CH), jnp.int32),
        pltpu.VMEM((CH, D), jnp.float32),
        pltpu.VMEM((CH, D), jnp.float32),
        pltpu.VMEM_SHARED((NP, D), jnp.float32),
        pltpu.SemaphoreType.DMA,
        pltpu.SemaphoreType.DMA,
        pltpu.SemaphoreType.DMA,
        pltpu.SemaphoreType.DMA,
        pltpu.SemaphoreType.DMA,
        pltpu.SemaphoreType.DMA,
        pltpu.SemaphoreType.DMA,
        pltpu.SemaphoreType.DMA,
    ],
)
def _hop_kernel(y_hbm, ei_hbm, tail_hbm, out_hbm,
                idxgw, idxsw, r0, r1, acc,
                ig0, ig1, is0, is1, gs0, gs1, ws0, ws1):
    c = lax.axis_index("c")
    s = lax.axis_index("s")
    wid = c * NS + s
    rows = (r0, r1)
    igs = (ig0, ig1)
    iss = (is0, is1)
    gss = (gs0, gs1)

    def ld_idx(which, j, dst, sem):
        q = wid * NCH + j

        @pl.when(q < EC)
        def _():
            pltpu.make_async_copy(ei_hbm.at[which, q], dst, sem).start()

        @pl.when(q >= EC)
        def _():
            pltpu.make_async_copy(tail_hbm.at[q - EC], dst, sem).start()

    def wt_idx(which, j, dst, sem):
        q = jnp.minimum(wid * NCH + j, EC - 1)
        pltpu.make_async_copy(ei_hbm.at[which, q], dst, sem).wait()

    for b in range(2):
        ld_idx(1, b, idxgw.at[b], igs[b])
        ld_idx(0, b, idxsw.at[b], iss[b])
    for i in range(CH):
        for k in range(D // 16):
            r0[i, pl.ds(k * 16, 16)] = jnp.zeros((16,), jnp.float32)
    for i in range(RT // CH):
        pltpu.make_async_copy(r0, acc.at[pl.ds(s * RT + i * CH, CH)],
                              ws0).start()
    for i in range(RT // CH):
        pltpu.make_async_copy(r0, acc.at[pl.ds(s * RT + i * CH, CH)],
                              ws0).wait()
    for b in range(2):
        wt_idx(1, b, idxgw.at[b], igs[b])
        pltpu.make_async_copy(y_hbm.at[idxgw.at[b]], rows[b], gss[b]).start()
    plsc.subcore_barrier()

    def body(t, carry):
        for b in range(2):
            j = t * 2 + b
            pltpu.make_async_copy(y_hbm.at[idxgw.at[b]], rows[b],
                                  gss[b]).wait()

            @pl.when(j + 2 < NCH)
            def _():
                ld_idx(1, j + 2, idxgw.at[b], igs[b])
            wt_idx(0, j, idxsw.at[b], iss[b])
            pltpu.sync_copy(rows[b], acc.at[idxsw.at[b]], add=True)

            @pl.when(j + 2 < NCH)
            def _():
                ld_idx(0, j + 2, idxsw.at[b], iss[b])
                wt_idx(1, j + 2, idxgw.at[b], igs[b])
                pltpu.make_async_copy(y_hbm.at[idxgw.at[b]], rows[b],
                                      gss[b]).start()
        return carry

    lax.fori_loop(0, NCH // 2, body, 0)
    for j in range((NCH // 2) * 2, NCH):
        b = j % 2
        pltpu.make_async_copy(y_hbm.at[idxgw.at[b]], rows[b], gss[b]).wait()
        wt_idx(0, j, idxsw.at[b], iss[b])
        pltpu.sync_copy(rows[b], acc.at[idxsw.at[b]], add=True)
    plsc.subcore_barrier()

    sts = (r0, r1)
    wss = (ws0, ws1)
    descs = [None, None]
    for i in range(RT // CH):
        b = i % 2
        if descs[b] is not None:
            descs[b].wait()
        pltpu.sync_copy(acc.at[pl.ds(s * RT + i * CH, CH)], sts[b])
        descs[b] = pltpu.async_copy(
            sts[b], out_hbm.at[c, pl.ds(s * RT + i * CH, CH)], wss[b])
    for d_ in descs:
        d_.wait()


def _norm_body(degp_ref, x_ref, y_ref, deg_ref):
    deg = degp_ref[0] + degp_ref[1]
    y_ref[...] = x_ref[...] * lax.rsqrt(deg)
    deg_ref[...] = deg


_norm = pl.pallas_call(
    _norm_body,
    grid=(NP // BN,),
    in_specs=[
        pl.BlockSpec((NC, BN, 1), lambda i: (0, i, 0)),
        pl.BlockSpec((BN, D), lambda i: (i, 0)),
    ],
    out_specs=[
        pl.BlockSpec((BN, D), lambda i: (i, 0)),
        pl.BlockSpec((BN, 1), lambda i: (i, 0)),
    ],
    out_shape=[
        jax.ShapeDtypeStruct((NP, D), jnp.float32),
        jax.ShapeDtypeStruct((NP, 1), jnp.float32),
    ],
)


def _scale_body(zp_ref, deg_ref, y_ref):
    y_ref[...] = (zp_ref[0] + zp_ref[1]) / deg_ref[...]


_scale = pl.pallas_call(
    _scale_body,
    grid=(NP // BN,),
    in_specs=[
        pl.BlockSpec((NC, BN, D), lambda i: (0, i, 0)),
        pl.BlockSpec((BN, 1), lambda i: (i, 0)),
    ],
    out_specs=pl.BlockSpec((BN, D), lambda i: (i, 0)),
    out_shape=jax.ShapeDtypeStruct((NP, D), jnp.float32),
)


def _final_body(zp_ref, deg_ref, w_ref, b_ref, o_ref):
    z = (zp_ref[0] + zp_ref[1]) * lax.rsqrt(deg_ref[...])
    o_ref[...] = lax.dot_general(
        z, w_ref[...], (((1,), (1,)), ((), ())),
        preferred_element_type=jnp.float32) + b_ref[...]


_final = pl.pallas_call(
    _final_body,
    grid=(N // BM,),
    in_specs=[
        pl.BlockSpec((NC, BM, D), lambda i: (0, i, 0)),
        pl.BlockSpec((BM, 1), lambda i: (i, 0)),
        pl.BlockSpec((D, D), lambda i: (0, 0)),
        pl.BlockSpec((1, D), lambda i: (0, 0)),
    ],
    out_specs=pl.BlockSpec((BM, D), lambda i: (i, 0)),
    out_shape=jax.ShapeDtypeStruct((N, D), jnp.float32),
)


@jax.jit
def kernel(x, edge_index, W, b):
    ei = edge_index.reshape(2, EC, CH)
    tail = jnp.asarray(_TAIL)

    degp = _deg_kernel(ei, tail)
    y, deg1 = _norm(degp.reshape(NC, NP, 1), x)
    for _ in range(HOPS - 1):
        zp = _hop_kernel(y, ei, tail)
        y = _scale(zp, deg1)
    zp = _hop_kernel(y, ei, tail)
    return _final(zp, deg1, W, b.reshape(1, D))


# speedup vs baseline: 1.3085x; 1.0071x over previous
import functools

import jax
import jax.numpy as jnp
from jax import lax
from jax.experimental import pallas as pl
from jax.experimental.pallas import tpu as pltpu
from jax.experimental.pallas import tpu_sc as plsc

N = 10000
NP = 10240
E = 320000
D = 128
HOPS = 3
NC = 2
NS = 16
NW = NC * NS
CH = 128
NCH = 81
EW = NCH * CH
EP = NW * EW
RT = NP // NS
EC = E // CH
TC_ = (EP - E) // CH
BN = 1024
BM = 1000

import numpy as _np
_TAIL = _np.concatenate([
    _np.arange(N, dtype=_np.int32),
    N + (_np.arange(EP - E - N, dtype=_np.int32) % (NP - N)),
]).reshape(TC_, CH)

_mesh = plsc.VectorSubcoreMesh(core_axis_name="c", subcore_axis_name="s")


@functools.partial(
    pl.kernel,
    out_type=jax.ShapeDtypeStruct((NC, NP), jnp.float32),
    mesh=_mesh,
    scratch_types=[
        pltpu.VMEM((4, CH), jnp.int32),
        pltpu.VMEM((CH,), jnp.float32),
        pltpu.VMEM((RT,), jnp.float32),
        pltpu.VMEM_SHARED((NP,), jnp.float32),
        pltpu.SemaphoreType.DMA,
        pltpu.SemaphoreType.DMA,
        pltpu.SemaphoreType.DMA,
        pltpu.SemaphoreType.DMA,
        pltpu.SemaphoreType.DMA,
        pltpu.SemaphoreType.DMA,
        pltpu.SemaphoreType.DMA,
        pltpu.SemaphoreType.DMA,
    ],
)
def _deg_kernel(ei_hbm, tail_hbm, out_hbm, idxw, ones_v, stage_v, acc,
                is0, is1, is2, is3, ss0, ss1, ss2, ss3):
    c = lax.axis_index("c")
    s = lax.axis_index("s")
    wid = c * NS + s
    iss = (is0, is1, is2, is3)
    sss = (ss0, ss1, ss2, ss3)

    def ld_col(j, dst, sem):
        q = wid * NCH + j

        @pl.when(q < EC)
        def _():
            pltpu.make_async_copy(ei_hbm.at[1, pl.ds(q * CH, CH)], dst,
                                  sem).start()

        @pl.when(q >= EC)
        def _():
            pltpu.make_async_copy(tail_hbm.at[q - EC], dst, sem).start()

    def wt_col(j, dst, sem):
        q = jnp.minimum(wid * NCH + j, EC - 1)
        pltpu.make_async_copy(ei_hbm.at[1, pl.ds(q * CH, CH)], dst,
                              sem).wait()
    for p in range(2):
        ld_col(p, idxw.at[p], iss[p])
    for i in range(CH // 16):
        ones_v[pl.ds(i * 16, 16)] = jnp.ones((16,), jnp.float32)
    for i in range(RT // 16):
        stage_v[pl.ds(i * 16, 16)] = jnp.zeros((16,), jnp.float32)
    pltpu.sync_copy(stage_v, acc.at[pl.ds(s * RT, RT)])
    plsc.subcore_barrier()

    NB = (NCH // 4) * 4

    def body(t, carry):
        for p in range(4):
            j = t * 4 + p
            wt_col(j, idxw.at[p], iss[p])
            pltpu.async_copy(ones_v, acc.at[idxw.at[p]], sss[p], add=True)
            pn = (p + 2) % 4

            @pl.when((j >= 2) & (j + 2 < NCH))
            def _():
                pltpu.make_async_copy(ones_v, acc.at[idxw.at[pn]],
                                      sss[pn]).wait()

            @pl.when(j + 2 < NCH)
            def _():
                ld_col(j + 2, idxw.at[pn], iss[pn])
        return carry

    lax.fori_loop(0, NB // 4, body, 0)
    for j in range(NB, NCH):
        p = j % 4
        wt_col(j, idxw.at[p], iss[p])
        pltpu.async_copy(ones_v, acc.at[idxw.at[p]], sss[p], add=True)
    for j in range(NCH - 4, NCH):
        p = j % 4
        pltpu.make_async_copy(ones_v, acc.at[idxw.at[p]], sss[p]).wait()
    plsc.subcore_barrier()
    pltpu.sync_copy(acc.at[pl.ds(s * RT, RT)], stage_v)
    pltpu.sync_copy(stage_v, out_hbm.at[c, pl.ds(s * RT, RT)])


@functools.partial(
    pl.kernel,
    out_type=jax.ShapeDtypeStruct((NC, NP, D), jnp.float32),
    mesh=_mesh,
    scratch_types=[
        pltpu.VMEM((2, CH), jnp.int32),
        pltpu.VMEM((2, CH), jnp.int32),
        pltpu.VMEM((CH, D), jnp.float32),
        pltpu.VMEM((CH, D), jnp.float32),
        pltpu.VMEM_SHARED((NP, D), jnp.float32),
        pltpu.SemaphoreType.DMA,
        pltpu.SemaphoreType.DMA,
        pltpu.SemaphoreType.DMA,
        pltpu.SemaphoreType.DMA,
        pltpu.SemaphoreType.DMA,
        pltpu.SemaphoreType.DMA,
        pltpu.SemaphoreType.DMA,
        pltpu.SemaphoreType.DMA,
    ],
)
def _hop_kernel(y_hbm, ei_hbm, tail_hbm, out_hbm,
                idxgw, idxsw, r0, r1, acc,
                ig0, ig1, is0, is1, gs0, gs1, ws0, ws1):
    c = lax.axis_index("c")
    s = lax.axis_index("s")
    wid = c * NS + s
    rows = (r0, r1)
    igs = (ig0, ig1)
    iss = (is0, is1)
    gss = (gs0, gs1)

    def ld_idx(which, j, dst, sem):
        q = wid * NCH + j

        @pl.when(q < EC)
        def _():
            pltpu.make_async_copy(ei_hbm.at[which, pl.ds(q * CH, CH)], dst,
                                  sem).start()

        @pl.when(q >= EC)
        def _():
            pltpu.make_async_copy(tail_hbm.at[q - EC], dst, sem).start()

    def wt_idx(which, j, dst, sem):
        q = jnp.minimum(wid * NCH + j, EC - 1)
        pltpu.make_async_copy(ei_hbm.at[which, pl.ds(q * CH, CH)], dst,
                              sem).wait()

    for b in range(2):
        ld_idx(1, b, idxgw.at[b], igs[b])
        ld_idx(0, b, idxsw.at[b], iss[b])
    for i in range(CH):
        for k in range(D // 16):
            r0[i, pl.ds(k * 16, 16)] = jnp.zeros((16,), jnp.float32)
    for i in range(RT // CH):
        pltpu.make_async_copy(r0, acc.at[pl.ds(s * RT + i * CH, CH)],
                              ws0).start()
    for i in range(RT // CH):
        pltpu.make_async_copy(r0, acc.at[pl.ds(s * RT + i * CH, CH)],
                              ws0).wait()
    for b in range(2):
        wt_idx(1, b, idxgw.at[b], igs[b])
        pltpu.make_async_copy(y_hbm.at[idxgw.at[b]], rows[b], gss[b]).start()
    plsc.subcore_barrier()

    def body(t, carry):
        for b in range(2):
            j = t * 2 + b
            pltpu.make_async_copy(y_hbm.at[idxgw.at[b]], rows[b],
                                  gss[b]).wait()

            @pl.when(j + 2 < NCH)
            def _():
                ld_idx(1, j + 2, idxgw.at[b], igs[b])
            wt_idx(0, j, idxsw.at[b], iss[b])
            pltpu.sync_copy(rows[b], acc.at[idxsw.at[b]], add=True)

            @pl.when(j + 2 < NCH)
            def _():
                ld_idx(0, j + 2, idxsw.at[b], iss[b])
                wt_idx(1, j + 2, idxgw.at[b], igs[b])
                pltpu.make_async_copy(y_hbm.at[idxgw.at[b]], rows[b],
                                      gss[b]).start()
        return carry

    lax.fori_loop(0, NCH // 2, body, 0)
    for j in range((NCH // 2) * 2, NCH):
        b = j % 2
        pltpu.make_async_copy(y_hbm.at[idxgw.at[b]], rows[b], gss[b]).wait()
        wt_idx(0, j, idxsw.at[b], iss[b])
        pltpu.sync_copy(rows[b], acc.at[idxsw.at[b]], add=True)
    plsc.subcore_barrier()

    sts = (r0, r1)
    wss = (ws0, ws1)
    descs = [None, None]
    for i in range(RT // CH):
        b = i % 2
        if descs[b] is not None:
            descs[b].wait()
        pltpu.sync_copy(acc.at[pl.ds(s * RT + i * CH, CH)], sts[b])
        descs[b] = pltpu.async_copy(
            sts[b], out_hbm.at[c, pl.ds(s * RT + i * CH, CH)], wss[b])
    for d_ in descs:
        d_.wait()


def _norm_body(degp_ref, x_ref, y_ref, deg_ref):
    deg = degp_ref[0] + degp_ref[1]
    y_ref[...] = x_ref[...] * lax.rsqrt(deg)
    deg_ref[...] = deg


_norm = pl.pallas_call(
    _norm_body,
    grid=(NP // BN,),
    in_specs=[
        pl.BlockSpec((NC, BN, 1), lambda i: (0, i, 0)),
        pl.BlockSpec((BN, D), lambda i: (i, 0)),
    ],
    out_specs=[
        pl.BlockSpec((BN, D), lambda i: (i, 0)),
        pl.BlockSpec((BN, 1), lambda i: (i, 0)),
    ],
    out_shape=[
        jax.ShapeDtypeStruct((NP, D), jnp.float32),
        jax.ShapeDtypeStruct((NP, 1), jnp.float32),
    ],
)


def _scale_body(zp_ref, deg_ref, y_ref):
    y_ref[...] = (zp_ref[0] + zp_ref[1]) / deg_ref[...]


_scale = pl.pallas_call(
    _scale_body,
    grid=(NP // BN,),
    in_specs=[
        pl.BlockSpec((NC, BN, D), lambda i: (0, i, 0)),
        pl.BlockSpec((BN, 1), lambda i: (i, 0)),
    ],
    out_specs=pl.BlockSpec((BN, D), lambda i: (i, 0)),
    out_shape=jax.ShapeDtypeStruct((NP, D), jnp.float32),
)


def _final_body(zp_ref, deg_ref, w_ref, b_ref, o_ref):
    z = (zp_ref[0] + zp_ref[1]) * lax.rsqrt(deg_ref[...])
    o_ref[...] = lax.dot_general(
        z, w_ref[...], (((1,), (1,)), ((), ())),
        preferred_element_type=jnp.float32) + b_ref[...]


_final = pl.pallas_call(
    _final_body,
    grid=(N // BM,),
    in_specs=[
        pl.BlockSpec((NC, BM, D), lambda i: (0, i, 0)),
        pl.BlockSpec((BM, 1), lambda i: (i, 0)),
        pl.BlockSpec((D, D), lambda i: (0, 0)),
        pl.BlockSpec((1, D), lambda i: (0, 0)),
    ],
    out_specs=pl.BlockSpec((BM, D), lambda i: (i, 0)),
    out_shape=jax.ShapeDtypeStruct((N, D), jnp.float32),
)


@jax.jit
def kernel(x, edge_index, W, b):
    ei = edge_index
    tail = jnp.asarray(_TAIL)

    degp = _deg_kernel(ei, tail)
    y, deg1 = _norm(degp.reshape(NC, NP, 1), x)
    for _ in range(HOPS - 1):
        zp = _hop_kernel(y, ei, tail)
        y = _scale(zp, deg1)
    zp = _hop_kernel(y, ei, tail)
    return _final(zp, deg1, W, b.reshape(1, D))


# speedup vs baseline: 1.3269x; 1.0141x over previous
import functools

import jax
import jax.numpy as jnp
from jax import lax
from jax.experimental import pallas as pl
from jax.experimental.pallas import tpu as pltpu
from jax.experimental.pallas import tpu_sc as plsc

N = 10000
NP = 10240
E = 320000
D = 128
HOPS = 3
NC = 2
NS = 16
NW = NC * NS
CH = 128
NCH = 81
EW = NCH * CH
EP = NW * EW
RT = NP // NS
EC = E // CH
TC_ = (EP - E) // CH
BN = 2048
BM = 1000

import numpy as _np
_TAIL = _np.concatenate([
    _np.arange(N, dtype=_np.int32),
    N + (_np.arange(EP - E - N, dtype=_np.int32) % (NP - N)),
]).reshape(TC_, CH)

_mesh = plsc.VectorSubcoreMesh(core_axis_name="c", subcore_axis_name="s")


@functools.partial(
    pl.kernel,
    out_type=jax.ShapeDtypeStruct((NC, NP), jnp.float32),
    mesh=_mesh,
    scratch_types=[
        pltpu.VMEM((4, CH), jnp.int32),
        pltpu.VMEM((CH,), jnp.float32),
        pltpu.VMEM((RT,), jnp.float32),
        pltpu.VMEM_SHARED((NP,), jnp.float32),
        pltpu.SemaphoreType.DMA,
        pltpu.SemaphoreType.DMA,
        pltpu.SemaphoreType.DMA,
        pltpu.SemaphoreType.DMA,
        pltpu.SemaphoreType.DMA,
        pltpu.SemaphoreType.DMA,
        pltpu.SemaphoreType.DMA,
        pltpu.SemaphoreType.DMA,
    ],
)
def _deg_kernel(ei_hbm, tail_hbm, out_hbm, idxw, ones_v, stage_v, acc,
                is0, is1, is2, is3, ss0, ss1, ss2, ss3):
    c = lax.axis_index("c")
    s = lax.axis_index("s")
    wid = c * NS + s
    iss = (is0, is1, is2, is3)
    sss = (ss0, ss1, ss2, ss3)

    def ld_col(j, dst, sem):
        q = wid * NCH + j

        @pl.when(q < EC)
        def _():
            pltpu.make_async_copy(ei_hbm.at[1, pl.ds(q * CH, CH)], dst,
                                  sem).start()

        @pl.when(q >= EC)
        def _():
            pltpu.make_async_copy(tail_hbm.at[q - EC], dst, sem).start()

    def wt_col(j, dst, sem):
        q = jnp.minimum(wid * NCH + j, EC - 1)
        pltpu.make_async_copy(ei_hbm.at[1, pl.ds(q * CH, CH)], dst,
                              sem).wait()
    for p in range(2):
        ld_col(p, idxw.at[p], iss[p])
    for i in range(CH // 16):
        ones_v[pl.ds(i * 16, 16)] = jnp.ones((16,), jnp.float32)
    for i in range(RT // 16):
        stage_v[pl.ds(i * 16, 16)] = jnp.zeros((16,), jnp.float32)
    pltpu.sync_copy(stage_v, acc.at[pl.ds(s * RT, RT)])
    plsc.subcore_barrier()

    NB = (NCH // 4) * 4

    def body(t, carry):
        for p in range(4):
            j = t * 4 + p
            wt_col(j, idxw.at[p], iss[p])
            pltpu.async_copy(ones_v, acc.at[idxw.at[p]], sss[p], add=True)
            pn = (p + 2) % 4

            @pl.when((j >= 2) & (j + 2 < NCH))
            def _():
                pltpu.make_async_copy(ones_v, acc.at[idxw.at[pn]],
                                      sss[pn]).wait()

            @pl.when(j + 2 < NCH)
            def _():
                ld_col(j + 2, idxw.at[pn], iss[pn])
        return carry

    lax.fori_loop(0, NB // 4, body, 0)
    for j in range(NB, NCH):
        p = j % 4
        wt_col(j, idxw.at[p], iss[p])
        pltpu.async_copy(ones_v, acc.at[idxw.at[p]], sss[p], add=True)
    for j in range(NCH - 4, NCH):
        p = j % 4
        pltpu.make_async_copy(ones_v, acc.at[idxw.at[p]], sss[p]).wait()
    plsc.subcore_barrier()
    pltpu.sync_copy(acc.at[pl.ds(s * RT, RT)], stage_v)
    pltpu.sync_copy(stage_v, out_hbm.at[c, pl.ds(s * RT, RT)])


@functools.partial(
    pl.kernel,
    out_type=jax.ShapeDtypeStruct((NC, NP, D), jnp.float32),
    mesh=_mesh,
    scratch_types=[
        pltpu.VMEM((2, CH), jnp.int32),
        pltpu.VMEM((2, CH), jnp.int32),
        pltpu.VMEM((CH, D), jnp.float32),
        pltpu.VMEM((CH, D), jnp.float32),
        pltpu.VMEM_SHARED((NP, D), jnp.float32),
        pltpu.SemaphoreType.DMA,
        pltpu.SemaphoreType.DMA,
        pltpu.SemaphoreType.DMA,
        pltpu.SemaphoreType.DMA,
        pltpu.SemaphoreType.DMA,
        pltpu.SemaphoreType.DMA,
        pltpu.SemaphoreType.DMA,
        pltpu.SemaphoreType.DMA,
    ],
)
def _hop_kernel(y_hbm, ei_hbm, tail_hbm, out_hbm,
                idxgw, idxsw, r0, r1, acc,
                ig0, ig1, is0, is1, gs0, gs1, ws0, ws1):
    c = lax.axis_index("c")
    s = lax.axis_index("s")
    wid = c * NS + s
    rows = (r0, r1)
    igs = (ig0, ig1)
    iss = (is0, is1)
    gss = (gs0, gs1)

    def ld_idx(which, j, dst, sem):
        q = wid * NCH + j

        @pl.when(q < EC)
        def _():
            pltpu.make_async_copy(ei_hbm.at[which, pl.ds(q * CH, CH)], dst,
                                  sem).start()

        @pl.when(q >= EC)
        def _():
            pltpu.make_async_copy(tail_hbm.at[q - EC], dst, sem).start()

    def wt_idx(which, j, dst, sem):
        q = jnp.minimum(wid * NCH + j, EC - 1)
        pltpu.make_async_copy(ei_hbm.at[which, pl.ds(q * CH, CH)], dst,
                              sem).wait()

    for b in range(2):
        ld_idx(1, b, idxgw.at[b], igs[b])
        ld_idx(0, b, idxsw.at[b], iss[b])
    for i in range(CH):
        for k in range(D // 16):
            r0[i, pl.ds(k * 16, 16)] = jnp.zeros((16,), jnp.float32)
    for i in range(RT // CH):
        pltpu.make_async_copy(r0, acc.at[pl.ds(s * RT + i * CH, CH)],
                              ws0).start()
    for i in range(RT // CH):
        pltpu.make_async_copy(r0, acc.at[pl.ds(s * RT + i * CH, CH)],
                              ws0).wait()
    for b in range(2):
        wt_idx(1, b, idxgw.at[b], igs[b])
        pltpu.make_async_copy(y_hbm.at[idxgw.at[b]], rows[b], gss[b]).start()
    plsc.subcore_barrier()

    def body(t, carry):
        for b in range(2):
            j = t * 2 + b
            pltpu.make_async_copy(y_hbm.at[idxgw.at[b]], rows[b],
                                  gss[b]).wait()

            @pl.when(j + 2 < NCH)
            def _():
                ld_idx(1, j + 2, idxgw.at[b], igs[b])
            wt_idx(0, j, idxsw.at[b], iss[b])
            pltpu.sync_copy(rows[b], acc.at[idxsw.at[b]], add=True)

            @pl.when(j + 2 < NCH)
            def _():
                ld_idx(0, j + 2, idxsw.at[b], iss[b])
                wt_idx(1, j + 2, idxgw.at[b], igs[b])
                pltpu.make_async_copy(y_hbm.at[idxgw.at[b]], rows[b],
                                      gss[b]).start()
        return carry

    lax.fori_loop(0, NCH // 2, body, 0)
    for j in range((NCH // 2) * 2, NCH):
        b = j % 2
        pltpu.make_async_copy(y_hbm.at[idxgw.at[b]], rows[b], gss[b]).wait()
        wt_idx(0, j, idxsw.at[b], iss[b])
        pltpu.sync_copy(rows[b], acc.at[idxsw.at[b]], add=True)
    plsc.subcore_barrier()

    sts = (r0, r1)
    wss = (ws0, ws1)
    descs = [None, None]
    for i in range(RT // CH):
        b = i % 2
        if descs[b] is not None:
            descs[b].wait()
        pltpu.sync_copy(acc.at[pl.ds(s * RT + i * CH, CH)], sts[b])
        descs[b] = pltpu.async_copy(
            sts[b], out_hbm.at[c, pl.ds(s * RT + i * CH, CH)], wss[b])
    for d_ in descs:
        d_.wait()


def _norm_body(degp_ref, x_ref, y_ref, deg_ref):
    deg = degp_ref[0] + degp_ref[1]
    y_ref[...] = x_ref[...] * lax.rsqrt(deg)
    deg_ref[...] = deg


_norm = pl.pallas_call(
    _norm_body,
    grid=(NP // BN,),
    in_specs=[
        pl.BlockSpec((NC, BN, 1), lambda i: (0, i, 0)),
        pl.BlockSpec((BN, D), lambda i: (i, 0)),
    ],
    out_specs=[
        pl.BlockSpec((BN, D), lambda i: (i, 0)),
        pl.BlockSpec((BN, 1), lambda i: (i, 0)),
    ],
    out_shape=[
        jax.ShapeDtypeStruct((NP, D), jnp.float32),
        jax.ShapeDtypeStruct((NP, 1), jnp.float32),
    ],
)


def _scale_body(zp_ref, deg_ref, y_ref):
    y_ref[...] = (zp_ref[0] + zp_ref[1]) / deg_ref[...]


_scale = pl.pallas_call(
    _scale_body,
    grid=(NP // BN,),
    in_specs=[
        pl.BlockSpec((NC, BN, D), lambda i: (0, i, 0)),
        pl.BlockSpec((BN, 1), lambda i: (i, 0)),
    ],
    out_specs=pl.BlockSpec((BN, D), lambda i: (i, 0)),
    out_shape=jax.ShapeDtypeStruct((NP, D), jnp.float32),
)


def _final_body(zp_ref, deg_ref, w_ref, b_ref, o_ref):
    z = (zp_ref[0] + zp_ref[1]) * lax.rsqrt(deg_ref[...])
    o_ref[...] = lax.dot_general(
        z, w_ref[...], (((1,), (1,)), ((), ())),
        preferred_element_type=jnp.float32) + b_ref[...]


_final = pl.pallas_call(
    _final_body,
    grid=(N // BM,),
    in_specs=[
        pl.BlockSpec((NC, BM, D), lambda i: (0, i, 0)),
        pl.BlockSpec((BM, 1), lambda i: (i, 0)),
        pl.BlockSpec((D, D), lambda i: (0, 0)),
        pl.BlockSpec((1, D), lambda i: (0, 0)),
    ],
    out_specs=pl.BlockSpec((BM, D), lambda i: (i, 0)),
    out_shape=jax.ShapeDtypeStruct((N, D), jnp.float32),
)


@jax.jit
def kernel(x, edge_index, W, b):
    ei = edge_index
    tail = jnp.asarray(_TAIL)

    degp = _deg_kernel(ei, tail)
    y, deg1 = _norm(degp.reshape(NC, NP, 1), x)
    for _ in range(HOPS - 1):
        zp = _hop_kernel(y, ei, tail)
        y = _scale(zp, deg1)
    zp = _hop_kernel(y, ei, tail)
    return _final(zp, deg1, W, b.reshape(1, D))
